# Initial kernel scaffold; baseline (speedup 1.0000x reference)
import jax
import jax.numpy as jnp
from jax.experimental import pallas as pl


def kernel(bag_encoding, V_w, V_b, U_w, U_b, w_w, w_b, dec_w, dec_b, batch_indices):
    raise NotImplementedError("write your pallas kernel here")


# speedup vs baseline: 4.3258x; 4.3258x over previous
import functools

import jax
import jax.numpy as jnp
from jax import lax
from jax.experimental import pallas as pl
from jax.experimental.pallas import tpu as pltpu
from jax.experimental.pallas import tpu_sc as plsc

_N = 320000
_D = 128
_H = 64
_NUM_BAGS = 10000

_BLK = 512
_NW = 32
_GRP = 128
_G_PER_W = 80
_NPAD = _NW * _G_PER_W * _GRP
_ACC_ROWS = 10240
_ROWS_PER_TILE = _ACC_ROWS // 16


def _alpha_z_body(x_ref, vw_ref, vb_ref, uw_ref, ub_ref, ww_ref, wb_ref,
                  p16_ref, e0_ref, out_ref):
    x = x_ref[...]
    q = jnp.tanh(
        lax.dot_general(x, vw_ref[...], (((1,), (1,)), ((), ())),
                        preferred_element_type=jnp.float32) + vb_ref[...])
    u = lax.dot_general(x, uw_ref[...], (((1,), (1,)), ((), ())),
                        preferred_element_type=jnp.float32) + ub_ref[...]
    g = 1.0 / (1.0 + jnp.exp(-u))
    s = jnp.sum((q * g) * ww_ref[...], axis=1, keepdims=True) + wb_ref[...]
    alpha = jnp.exp(s)
    z16 = lax.dot_general(x, p16_ref[...], (((1,), (1,)), ((), ())),
                          preferred_element_type=jnp.float32) + e0_ref[...]
    out_ref[...] = alpha * z16


def _seg_sum_body(w_hbm, idx_hbm, out_hbm, acc, wbuf, idxbuf):
    c = lax.axis_index("c")
    s = lax.axis_index("s")
    wid = s * 2 + c
    base_g = wid * _G_PER_W

    zero16 = jnp.zeros((16,), jnp.float32)

    def _z(i, carry):
        wbuf[i, :] = zero16
        return carry

    lax.fori_loop(0, 512, _z, 0)
    pltpu.sync_copy(wbuf.at[pl.ds(0, 512)],
                    acc.at[pl.ds(s * _ROWS_PER_TILE, 512)])
    pltpu.sync_copy(wbuf.at[pl.ds(0, _ROWS_PER_TILE - 512)],
                    acc.at[pl.ds(s * _ROWS_PER_TILE + 512,
                                 _ROWS_PER_TILE - 512)])
    plsc.subcore_barrier()

    for ci in range(_G_PER_W // 8):
        g0 = base_g + ci * 8
        pltpu.sync_copy(idx_hbm.at[pl.ds(g0, 8), :], idxbuf)
        for half in range(2):
            gh = g0 + half * 4
            pltpu.sync_copy(w_hbm.at[pl.ds(gh * _GRP, 4 * _GRP), :],
                            wbuf.at[pl.ds(0, 4 * _GRP)])
            for j in range(4):
                pltpu.sync_copy(wbuf.at[pl.ds(j * _GRP, _GRP)],
                                acc.at[idxbuf.at[half * 4 + j]], add=True)
    plsc.subcore_barrier()

    pltpu.sync_copy(acc.at[pl.ds(s * _ROWS_PER_TILE, _ROWS_PER_TILE)],
                    out_hbm.at[c, pl.ds(s * _ROWS_PER_TILE, _ROWS_PER_TILE), :])


def _head_body(p_ref, db_ref, out_ref):
    p = p_ref[0] + p_ref[1]
    a = p[:, 0:1]
    safe_a = jnp.where(a > 0, a, 1.0)
    ratio = jnp.where(a > 0, 1.0 / safe_a, 0.0)
    logits = p[:, 1:3] * ratio + db_ref[...]
    m = jnp.max(logits, axis=1, keepdims=True)
    e = jnp.exp(logits - m)
    sm = e / jnp.sum(e, axis=1, keepdims=True)
    out_ref[...] = sm[0:_NUM_BAGS, :]


def _make_seg_kernel():
    mesh = plsc.VectorSubcoreMesh(core_axis_name="c", subcore_axis_name="s")
    return functools.partial(
        pl.kernel,
        out_type=jax.ShapeDtypeStruct((2, _ACC_ROWS, 16), jnp.float32),
        mesh=mesh,
        compiler_params=pltpu.CompilerParams(use_tc_tiling_on_sc=False),
        scratch_types=[
            pltpu.VMEM_SHARED((_ACC_ROWS, 16), jnp.float32),
            pltpu.VMEM((4 * _GRP, 16), jnp.float32),
            pltpu.VMEM((8, _GRP), jnp.int32),
        ],
    )(_seg_sum_body)


def kernel(bag_encoding, V_w, V_b, U_w, U_b, w_w, w_b, dec_w, dec_b,
           batch_indices):
    n, d = bag_encoding.shape
    h = V_w.shape[0]

    p16 = jnp.zeros((16, d), jnp.float32).at[1:3, :].set(dec_w)
    e0 = jnp.zeros((1, 16), jnp.float32).at[0, 0].set(1.0)

    grid = n // _BLK
    w_rows = pl.pallas_call(
        _alpha_z_body,
        grid=(grid,),
        in_specs=[
            pl.BlockSpec((_BLK, d), lambda i: (i, 0)),
            pl.BlockSpec((h, d), lambda i: (0, 0)),
            pl.BlockSpec((1, h), lambda i: (0, 0)),
            pl.BlockSpec((h, d), lambda i: (0, 0)),
            pl.BlockSpec((1, h), lambda i: (0, 0)),
            pl.BlockSpec((1, h), lambda i: (0, 0)),
            pl.BlockSpec((1, 1), lambda i: (0, 0)),
            pl.BlockSpec((16, d), lambda i: (0, 0)),
            pl.BlockSpec((1, 16), lambda i: (0, 0)),
        ],
        out_specs=pl.BlockSpec((_BLK, 16), lambda i: (i, 0)),
        out_shape=jax.ShapeDtypeStruct((_NPAD, 16), jnp.float32),
    )(bag_encoding, V_w, V_b.reshape(1, h), U_w, U_b.reshape(1, h),
      w_w.reshape(1, h), w_b.reshape(1, 1), p16, e0)

    idx = batch_indices.astype(jnp.int32)
    idx_pad = jnp.concatenate(
        [idx, jnp.full((_NPAD - n,), _ACC_ROWS - 1, jnp.int32)])
    idx2 = idx_pad.reshape(_NPAD // _GRP, _GRP)

    partials = _make_seg_kernel()(w_rows, idx2)

    out = pl.pallas_call(
        _head_body,
        in_specs=[
            pl.BlockSpec((2, _ACC_ROWS, 16), lambda: (0, 0, 0)),
            pl.BlockSpec((1, 2), lambda: (0, 0)),
        ],
        out_specs=pl.BlockSpec((_NUM_BAGS, 2), lambda: (0, 0)),
        out_shape=jax.ShapeDtypeStruct((_NUM_BAGS, 2), jnp.float32),
    )(partials, dec_b.reshape(1, 2))
    return out


# speedup vs baseline: 8.4623x; 1.9562x over previous
import functools

import jax
import jax.numpy as jnp
from jax import lax
from jax.experimental import pallas as pl
from jax.experimental.pallas import tpu as pltpu
from jax.experimental.pallas import tpu_sc as plsc

_N = 320000
_D = 128
_H = 64
_NUM_BAGS = 10000

_BLK = 1280
_NW = 32
_GRP = 128
_G_PER_W = 80
_NPAD = _NW * _G_PER_W * _GRP
_ACC_ROWS = 10240
_ROWS_PER_TILE = _ACC_ROWS // 16


def _alpha_z_body(x_ref, vw_ref, vb_ref, uw_ref, ub_ref, ww16_ref, wb16_ref,
                  p16_ref, e0_ref, out_ref):
    x = x_ref[...]
    q = jnp.tanh(
        lax.dot_general(x, vw_ref[...], (((1,), (1,)), ((), ())),
                        preferred_element_type=jnp.float32) + vb_ref[...])
    u = lax.dot_general(x, uw_ref[...], (((1,), (1,)), ((), ())),
                        preferred_element_type=jnp.float32) + ub_ref[...]
    g = 1.0 / (1.0 + jnp.exp(-u))
    s16 = lax.dot_general(q * g, ww16_ref[...], (((1,), (0,)), ((), ())),
                          preferred_element_type=jnp.float32) + wb16_ref[...]
    alpha16 = jnp.exp(s16)
    z16 = lax.dot_general(x, p16_ref[...], (((1,), (1,)), ((), ())),
                          preferred_element_type=jnp.float32) + e0_ref[...]
    y = alpha16 * z16
    g8 = _BLK // 8
    for a in range(8):
        out_ref[:, 16 * a:16 * (a + 1)] = y[g8 * a:g8 * (a + 1), :]


def _seg_sum_body(w_hbm, idx_hbm, out_hbm, acc, wbuf, idxbuf):
    c = lax.axis_index("c")
    s = lax.axis_index("s")
    wid = s * 2 + c
    base_g = wid * _G_PER_W

    zero16 = jnp.zeros((16,), jnp.float32)

    def _z(i, carry):
        wbuf[i, :] = zero16
        return carry

    lax.fori_loop(0, 512, _z, 0)
    pltpu.sync_copy(wbuf.at[pl.ds(0, 512)],
                    acc.at[pl.ds(s * _ROWS_PER_TILE, 512)])
    pltpu.sync_copy(wbuf.at[pl.ds(0, _ROWS_PER_TILE - 512)],
                    acc.at[pl.ds(s * _ROWS_PER_TILE + 512,
                                 _ROWS_PER_TILE - 512)])
    plsc.subcore_barrier()

    for ci in range(_G_PER_W // 8):
        g0 = base_g + ci * 8
        pltpu.sync_copy(idx_hbm.at[pl.ds(g0, 8), :], idxbuf)
        for half in range(2):
            gh = g0 + half * 4
            pltpu.sync_copy(w_hbm.at[pl.ds(gh * _GRP, 4 * _GRP), :],
                            wbuf.at[pl.ds(0, 4 * _GRP)])
            for j in range(4):
                pltpu.sync_copy(wbuf.at[pl.ds(j * _GRP, _GRP)],
                                acc.at[idxbuf.at[half * 4 + j]], add=True)
    plsc.subcore_barrier()

    pltpu.sync_copy(acc.at[pl.ds(s * _ROWS_PER_TILE, _ROWS_PER_TILE)],
                    out_hbm.at[c, pl.ds(s * _ROWS_PER_TILE, _ROWS_PER_TILE), :])


def _head_body(p_ref, db_ref, out_ref):
    p = p_ref[0] + p_ref[1]
    a = p[:, 0:1]
    safe_a = jnp.where(a > 0, a, 1.0)
    ratio = jnp.where(a > 0, 1.0 / safe_a, 0.0)
    logits = p[:, 1:3] * ratio + db_ref[...]
    m = jnp.max(logits, axis=1, keepdims=True)
    e = jnp.exp(logits - m)
    sm = e / jnp.sum(e, axis=1, keepdims=True)
    out_ref[...] = sm[0:_NUM_BAGS, :]


def _make_seg_kernel():
    mesh = plsc.VectorSubcoreMesh(core_axis_name="c", subcore_axis_name="s")
    return functools.partial(
        pl.kernel,
        out_type=jax.ShapeDtypeStruct((2, _ACC_ROWS, 16), jnp.float32),
        mesh=mesh,
        compiler_params=pltpu.CompilerParams(use_tc_tiling_on_sc=False),
        scratch_types=[
            pltpu.VMEM_SHARED((_ACC_ROWS, 16), jnp.float32),
            pltpu.VMEM((4 * _GRP, 16), jnp.float32),
            pltpu.VMEM((8, _GRP), jnp.int32),
        ],
    )(_seg_sum_body)


def kernel(bag_encoding, V_w, V_b, U_w, U_b, w_w, w_b, dec_w, dec_b,
           batch_indices):
    n, d = bag_encoding.shape
    h = V_w.shape[0]

    ww16 = jnp.tile(w_w.reshape(h, 1), (1, 16))
    wb16 = jnp.broadcast_to(w_b.reshape(1, 1), (1, 16))
    p16 = jnp.zeros((16, d), jnp.float32).at[1:3, :].set(dec_w)
    e0 = jnp.zeros((1, 16), jnp.float32).at[0, 0].set(1.0)

    grid = n // _BLK
    w_packed = pl.pallas_call(
        _alpha_z_body,
        grid=(grid,),
        in_specs=[
            pl.BlockSpec((_BLK, d), lambda i: (i, 0)),
            pl.BlockSpec((h, d), lambda i: (0, 0)),
            pl.BlockSpec((1, h), lambda i: (0, 0)),
            pl.BlockSpec((h, d), lambda i: (0, 0)),
            pl.BlockSpec((1, h), lambda i: (0, 0)),
            pl.BlockSpec((h, 16), lambda i: (0, 0)),
            pl.BlockSpec((1, 16), lambda i: (0, 0)),
            pl.BlockSpec((16, d), lambda i: (0, 0)),
            pl.BlockSpec((1, 16), lambda i: (0, 0)),
        ],
        out_specs=pl.BlockSpec((_BLK // 8, 128), lambda i: (i, 0)),
        out_shape=jax.ShapeDtypeStruct((_NPAD // 8, 128), jnp.float32),
    )(bag_encoding, V_w, V_b.reshape(1, h), U_w, U_b.reshape(1, h),
      ww16, wb16, p16, e0)
    w_rows = w_packed.reshape(_NPAD, 16)

    idx = batch_indices.astype(jnp.int32)
    idx_pad = jnp.concatenate(
        [idx, jnp.full((_NPAD - n,), _ACC_ROWS - 1, jnp.int32)])
    idx_rec = idx_pad.reshape(-1, 8, _BLK // 8).swapaxes(1, 2)
    idx2 = idx_rec.reshape(_NPAD // _GRP, _GRP)

    partials = _make_seg_kernel()(w_rows, idx2)

    out = pl.pallas_call(
        _head_body,
        in_specs=[
            pl.BlockSpec((2, _ACC_ROWS, 16), lambda: (0, 0, 0)),
            pl.BlockSpec((1, 2), lambda: (0, 0)),
        ],
        out_specs=pl.BlockSpec((_NUM_BAGS, 2), lambda: (0, 0)),
        out_shape=jax.ShapeDtypeStruct((_NUM_BAGS, 2), jnp.float32),
    )(partials, dec_b.reshape(1, 2))
    return out
